# baseline (speedup 1.0000x reference)
import functools

import jax
import jax.numpy as jnp
from jax import lax
from jax.experimental import pallas as pl
from jax.experimental.pallas import tpu as pltpu
from jax.experimental.pallas import tpu_sc as plsc

_TIME_SCALE = 288.0
_DAY_SCALE = 7.0


@functools.lru_cache(maxsize=None)
def _build_sc_lookup(B, T, N, C, D, Vt, Vd):
    info = plsc.get_sparse_core_info()
    NC, NS, L = info.num_cores, info.num_subcores, info.num_lanes
    NW = NC * NS
    assert B % NW == 0 and N % L == 0 and D % L == 0
    RPW = B // NW
    G = N // L
    ND = N * D
    ROW = T * N * C
    LAST = (T - 1) * N * C

    mesh = plsc.VectorSubcoreMesh(core_axis_name="c", subcore_axis_name="s")

    @functools.partial(
        pl.kernel,
        out_type=(
            jax.ShapeDtypeStruct((B, ND), jnp.float32),
            jax.ShapeDtypeStruct((B, ND), jnp.float32),
        ),
        mesh=mesh,
        compiler_params=pltpu.CompilerParams(
            use_tc_tiling_on_sc=False, needs_layout_passes=False),
        scratch_types=[
            pltpu.VMEM((Vt * D,), jnp.float32),
            pltpu.VMEM((Vd * D,), jnp.float32),
            pltpu.VMEM((2, N * C), jnp.float32),
            pltpu.VMEM((2, ND), jnp.float32),
            pltpu.VMEM((2, ND), jnp.float32),
            pltpu.SemaphoreType.DMA,
            pltpu.SemaphoreType.DMA,
            pltpu.SemaphoreType.DMA,
        ],
    )
    def k(hist_hbm, ttab_hbm, dtab_hbm, out_t_hbm, out_d_hbm,
          ttab_v, dtab_v, craw_v, slab_t, slab_d, csem, osem0, osem1):
        cid = lax.axis_index("c")
        sid = lax.axis_index("s")
        wid = sid * NC + cid
        base = wid * RPW
        lane = lax.iota(jnp.int32, L)
        laneD = lane * D
        laneC = lane * C

        pltpu.sync_copy(ttab_hbm, ttab_v)
        pltpu.sync_copy(dtab_hbm, dtab_v)
        pltpu.async_copy(
            hist_hbm.at[pl.ds(base * ROW + LAST, N * C)], craw_v.at[0], csem)

        def wait_out(buf, row):
            def mk(sem):
                pltpu.make_async_copy(
                    slab_t.at[0], out_t_hbm.at[row], sem).wait()
                pltpu.make_async_copy(
                    slab_d.at[0], out_d_hbm.at[row], sem).wait()

            @pl.when(buf == 0)
            def _():
                mk(osem0)

            @pl.when(buf == 1)
            def _():
                mk(osem1)

        def issue_out(buf, row):
            def issue(sem):
                pltpu.async_copy(slab_t.at[buf], out_t_hbm.at[row], sem)
                pltpu.async_copy(slab_d.at[buf], out_d_hbm.at[row], sem)

            @pl.when(buf == 0)
            def _():
                issue(osem0)

            @pl.when(buf == 1)
            def _():
                issue(osem1)

        def body(i, carry):
            buf = lax.rem(i, 2)
            nbuf = lax.rem(i + 1, 2)
            b = base + i

            pltpu.make_async_copy(
                hist_hbm.at[pl.ds(0, N * C)], craw_v.at[0], csem).wait()

            @pl.when(i + 1 < RPW)
            def _():
                pltpu.async_copy(
                    hist_hbm.at[pl.ds((b + 1) * ROW + LAST, N * C)],
                    craw_v.at[nbuf], csem)

            @pl.when(i >= 2)
            def _():
                wait_out(buf, b - 2)

            craw = craw_v.at[buf]
            st = slab_t.at[buf]
            sd = slab_d.at[buf]

            @plsc.parallel_loop(0, G)
            def group_body(g):
                off = g * L
                rowsC = laneC + off * C
                v1 = plsc.load_gather(craw, [rowsC + 1])
                v2 = plsc.load_gather(craw, [rowsC + 2])
                ti = (v1 * _TIME_SCALE).astype(jnp.int32) * D
                di = (v2 * _DAY_SCALE).astype(jnp.int32) * D
                n32 = laneD + off * D
                for d in range(D):
                    dd = (lane + d) & (D - 1)
                    tv = plsc.load_gather(ttab_v, [ti + dd])
                    plsc.store_scatter(st, [n32 + dd], tv)
                    dv = plsc.load_gather(dtab_v, [di + dd])
                    plsc.store_scatter(sd, [n32 + dd], dv)

            issue_out(buf, b)
            return carry

        lax.fori_loop(0, RPW, body, 0)
        wait_out(jnp.int32(0), base + RPW - 2)
        wait_out(jnp.int32(1), base + RPW - 1)

    return k


def kernel(history_data, time_in_day_emb, day_in_week_emb):
    B, T, N, C = history_data.shape
    Vt, D = time_in_day_emb.shape
    Vd, _ = day_in_week_emb.shape
    k = _build_sc_lookup(B, T, N, C, D, Vt, Vd)
    out_t, out_d = k(history_data.reshape(-1), time_in_day_emb.reshape(-1),
                     day_in_week_emb.reshape(-1))
    return (out_t.reshape(B, N, D), out_d.reshape(B, N, D))


# speedup vs baseline: 48.0324x; 48.0324x over previous
import functools

import jax
import jax.numpy as jnp
from jax import lax
from jax.experimental import pallas as pl
from jax.experimental.pallas import tpu as pltpu
from jax.experimental.pallas import tpu_sc as plsc

_TIME_SCALE = 288.0
_DAY_SCALE = 7.0


@functools.lru_cache(maxsize=None)
def _build_sc_lookup(B, N, D, Vt, Vd):
    info = plsc.get_sparse_core_info()
    NC, NS, L = info.num_cores, info.num_subcores, info.num_lanes
    NW = NC * NS
    assert B % NW == 0 and N % L == 0 and D % L == 0
    RPW = B // NW
    G = N // L

    mesh = plsc.VectorSubcoreMesh(core_axis_name="c", subcore_axis_name="s")

    @functools.partial(
        pl.kernel,
        out_type=(
            jax.ShapeDtypeStruct((B, N * D), jnp.float32),
            jax.ShapeDtypeStruct((B, N * D), jnp.float32),
        ),
        mesh=mesh,
        compiler_params=pltpu.CompilerParams(
            use_tc_tiling_on_sc=False, needs_layout_passes=False),
        scratch_types=[
            pltpu.VMEM((Vt * D,), jnp.float32),
            pltpu.VMEM((Vd * D,), jnp.float32),
            pltpu.VMEM((RPW, N), jnp.float32),
            pltpu.VMEM((RPW, N), jnp.float32),
            pltpu.VMEM((2, N * D), jnp.float32),
            pltpu.VMEM((2, N * D), jnp.float32),
            pltpu.SemaphoreType.DMA,
            pltpu.SemaphoreType.DMA,
        ],
    )
    def k(ch1_hbm, ch2_hbm, ttab_hbm, dtab_hbm, out_t_hbm, out_d_hbm,
          ttab_v, dtab_v, c1_v, c2_v, slab_t, slab_d, osem0, osem1):
        cid = lax.axis_index("c")
        sid = lax.axis_index("s")
        wid = sid * NC + cid
        base = wid * RPW
        lane = lax.iota(jnp.int32, L)
        laneD = lane * D

        pltpu.sync_copy(ttab_hbm, ttab_v)
        pltpu.sync_copy(dtab_hbm, dtab_v)
        pltpu.sync_copy(ch1_hbm.at[pl.ds(base, RPW)], c1_v)
        pltpu.sync_copy(ch2_hbm.at[pl.ds(base, RPW)], c2_v)

        def wait_out(buf, row):
            def mk(sem):
                pltpu.make_async_copy(
                    slab_t.at[0], out_t_hbm.at[row], sem).wait()
                pltpu.make_async_copy(
                    slab_d.at[0], out_d_hbm.at[row], sem).wait()

            @pl.when(buf == 0)
            def _():
                mk(osem0)

            @pl.when(buf == 1)
            def _():
                mk(osem1)

        def issue_out(buf, row):
            def issue(sem):
                pltpu.async_copy(slab_t.at[buf], out_t_hbm.at[row], sem)
                pltpu.async_copy(slab_d.at[buf], out_d_hbm.at[row], sem)

            @pl.when(buf == 0)
            def _():
                issue(osem0)

            @pl.when(buf == 1)
            def _():
                issue(osem1)

        def body(i, carry):
            buf = lax.rem(i, 2)
            b = base + i

            @pl.when(i >= 2)
            def _():
                wait_out(buf, b - 2)

            st = slab_t.at[buf]
            sd = slab_d.at[buf]

            @plsc.parallel_loop(0, G)
            def group_body(g):
                off = g * L
                v1 = c1_v[i, pl.ds(off, L)]
                v2 = c2_v[i, pl.ds(off, L)]
                ti = (v1 * _TIME_SCALE).astype(jnp.int32) * D
                di = (v2 * _DAY_SCALE).astype(jnp.int32) * D
                n32 = laneD + off * D
                for d in range(D):
                    dd = (lane + d) & (D - 1)
                    tv = plsc.load_gather(ttab_v, [ti + dd])
                    plsc.store_scatter(st, [n32 + dd], tv)
                    dv = plsc.load_gather(dtab_v, [di + dd])
                    plsc.store_scatter(sd, [n32 + dd], dv)

            issue_out(buf, b)
            return carry

        lax.fori_loop(0, RPW, body, 0)
        wait_out(jnp.int32(0), base + RPW - 2)
        wait_out(jnp.int32(1), base + RPW - 1)

    return k


def kernel(history_data, time_in_day_emb, day_in_week_emb):
    B, T, N, C = history_data.shape
    Vt, D = time_in_day_emb.shape
    Vd, _ = day_in_week_emb.shape
    ch1 = history_data[:, -1, :, 1]
    ch2 = history_data[:, -1, :, 2]
    k = _build_sc_lookup(B, N, D, Vt, Vd)
    out_t, out_d = k(ch1, ch2, time_in_day_emb.reshape(-1),
                     day_in_week_emb.reshape(-1))
    return (out_t.reshape(B, N, D), out_d.reshape(B, N, D))


# speedup vs baseline: 48.2923x; 1.0054x over previous
import functools

import jax
import jax.numpy as jnp
from jax import lax
from jax.experimental import pallas as pl
from jax.experimental.pallas import tpu as pltpu
from jax.experimental.pallas import tpu_sc as plsc

_TIME_SCALE = 288.0
_DAY_SCALE = 7.0


@functools.lru_cache(maxsize=None)
def _build_sc_lookup(B, N, D, Vt, Vd):
    info = plsc.get_sparse_core_info()
    NC, NS, L = info.num_cores, info.num_subcores, info.num_lanes
    NW = NC * NS
    assert B % NW == 0 and N % L == 0 and D % L == 0
    RPW = B // NW
    G = N // L

    mesh = plsc.VectorSubcoreMesh(core_axis_name="c", subcore_axis_name="s")

    @functools.partial(
        pl.kernel,
        out_type=(
            jax.ShapeDtypeStruct((B, N * D), jnp.float32),
            jax.ShapeDtypeStruct((B, N * D), jnp.float32),
        ),
        mesh=mesh,
        compiler_params=pltpu.CompilerParams(
            use_tc_tiling_on_sc=False, needs_layout_passes=False),
        scratch_types=[
            pltpu.VMEM((Vt * D,), jnp.float32),
            pltpu.VMEM((Vd * D,), jnp.float32),
            pltpu.VMEM((RPW, 2 * N), jnp.float32),
            pltpu.VMEM((2, N * D), jnp.float32),
            pltpu.VMEM((2, N * D), jnp.float32),
            pltpu.SemaphoreType.DMA,
            pltpu.SemaphoreType.DMA,
        ],
    )
    def k(ch_hbm, ttab_hbm, dtab_hbm, out_t_hbm, out_d_hbm,
          ttab_v, dtab_v, c_v, slab_t, slab_d, osem0, osem1):
        cid = lax.axis_index("c")
        sid = lax.axis_index("s")
        wid = sid * NC + cid
        base = wid * RPW
        lane = lax.iota(jnp.int32, L)
        laneD = lane * D

        pltpu.sync_copy(ttab_hbm, ttab_v)
        pltpu.sync_copy(dtab_hbm, dtab_v)
        pltpu.sync_copy(ch_hbm.at[pl.ds(base, RPW)], c_v)

        def wait_out(buf, row):
            def mk(sem):
                pltpu.make_async_copy(
                    slab_t.at[0], out_t_hbm.at[row], sem).wait()
                pltpu.make_async_copy(
                    slab_d.at[0], out_d_hbm.at[row], sem).wait()

            @pl.when(buf == 0)
            def _():
                mk(osem0)

            @pl.when(buf == 1)
            def _():
                mk(osem1)

        def issue_out(buf, row):
            def issue(sem):
                pltpu.async_copy(slab_t.at[buf], out_t_hbm.at[row], sem)
                pltpu.async_copy(slab_d.at[buf], out_d_hbm.at[row], sem)

            @pl.when(buf == 0)
            def _():
                issue(osem0)

            @pl.when(buf == 1)
            def _():
                issue(osem1)

        def body(i, carry):
            buf = lax.rem(i, 2)
            b = base + i

            @pl.when(i >= 2)
            def _():
                wait_out(buf, b - 2)

            st = slab_t.at[buf]
            sd = slab_d.at[buf]

            @plsc.parallel_loop(0, G)
            def group_body(g):
                off = g * L
                v1 = c_v[i, pl.ds(off, L)]
                v2 = c_v[i, pl.ds(N + off, L)]
                ti = (v1 * _TIME_SCALE).astype(jnp.int32) * D
                di = (v2 * _DAY_SCALE).astype(jnp.int32) * D
                n32 = laneD + off * D
                for d in range(D):
                    dd = (lane + d) & (D - 1)
                    tv = plsc.load_gather(ttab_v, [ti + dd])
                    plsc.store_scatter(st, [n32 + dd], tv)
                    dv = plsc.load_gather(dtab_v, [di + dd])
                    plsc.store_scatter(sd, [n32 + dd], dv)

            issue_out(buf, b)
            return carry

        lax.fori_loop(0, RPW, body, 0)
        wait_out(jnp.int32(0), base + RPW - 2)
        wait_out(jnp.int32(1), base + RPW - 1)

    return k


def kernel(history_data, time_in_day_emb, day_in_week_emb):
    B, T, N, C = history_data.shape
    Vt, D = time_in_day_emb.shape
    Vd, _ = day_in_week_emb.shape
    ch = jnp.concatenate(
        [history_data[:, -1, :, 1], history_data[:, -1, :, 2]], axis=1)
    k = _build_sc_lookup(B, N, D, Vt, Vd)
    out_t, out_d = k(ch, time_in_day_emb.reshape(-1),
                     day_in_week_emb.reshape(-1))
    return (out_t.reshape(B, N, D), out_d.reshape(B, N, D))
